# initial kernel scaffold (unmeasured)
import jax
import jax.numpy as jnp
from jax import lax
from jax.experimental import pallas as pl
from jax.experimental.pallas import tpu as pltpu

N_DEV = 4
B_LOC = 2
SQ = 512
SKV = 512
HG = 8
DH = 64
D_MODEL = 768
D_HID = HG * DH

_ANY = getattr(pltpu, "ANY", None)
if _ANY is None:
    _ANY = pltpu.MemorySpace.ANY
_CP = getattr(pltpu, "CompilerParams", None)
if _CP is None:
    _CP = pltpu.TPUCompilerParams


def kernel(x, Wq, K_ext, V_ext, Wo):
    def body(x_ref, wq_ref, k_ref, v_ref, wo_ref, out_ref,
             wq_all, wo_all, kbuf, vbuf, ctx_buf,
             wq_send, wq_recv, wo_send, wo_recv, kv_sem):
        me = lax.axis_index("i")
        right = lax.rem(me + 1, N_DEV)
        left = lax.rem(me + N_DEV - 1, N_DEV)

        barrier = pltpu.get_barrier_semaphore()
        pl.semaphore_signal(barrier, inc=1, device_id=(left,),
                            device_id_type=pl.DeviceIdType.MESH)
        pl.semaphore_signal(barrier, inc=1, device_id=(right,),
                            device_id_type=pl.DeviceIdType.MESH)
        pl.semaphore_wait(barrier, 2)

        wq_all[0, :, :] = wq_ref[:, :].astype(jnp.bfloat16)
        wo_all[0, :, :] = wo_ref[:, :].astype(jnp.bfloat16)

        x16 = x_ref[:, :, :].astype(jnp.bfloat16)

        row = lax.broadcasted_iota(jnp.int32, (SQ, SKV), 0)
        col = lax.broadcasted_iota(jnp.int32, (SQ, SKV), 1)
        bias = jnp.where((row // 64) % 4 == (col // 64) % 4, 0.0, -1e9)
        bias = bias.astype(jnp.float32)

        for s in range(N_DEV):
            if s < N_DEV - 1:
                rdma_wq = pltpu.make_async_remote_copy(
                    src_ref=wq_all.at[s], dst_ref=wq_all.at[s + 1],
                    send_sem=wq_send.at[s], recv_sem=wq_recv.at[s],
                    device_id=(right,), device_id_type=pl.DeviceIdType.MESH)
                rdma_wo = pltpu.make_async_remote_copy(
                    src_ref=wo_all.at[s], dst_ref=wo_all.at[s + 1],
                    send_sem=wo_send.at[s], recv_sem=wo_recv.at[s],
                    device_id=(right,), device_id_type=pl.DeviceIdType.MESH)
                rdma_wq.start()
                rdma_wo.start()

            o = lax.rem(me - s + N_DEV, N_DEV)
            cp_k = pltpu.make_async_copy(
                k_ref.at[pl.ds(me * B_LOC, B_LOC), :, pl.ds(o * HG, HG), :],
                kbuf, kv_sem.at[0])
            cp_v = pltpu.make_async_copy(
                v_ref.at[pl.ds(me * B_LOC, B_LOC), :, pl.ds(o * HG, HG), :],
                vbuf, kv_sem.at[1])
            cp_k.start()
            cp_v.start()
            cp_k.wait()
            cp_v.wait()

            wq_g = wq_all[s]
            wo_g = wo_all[s]
            for b in range(B_LOC):
                q16 = lax.dot(x16[b], wq_g,
                              preferred_element_type=jnp.float32
                              ).astype(jnp.bfloat16)
                kb = kbuf[b].reshape(SKV, HG * DH).astype(jnp.bfloat16)
                vb = vbuf[b].reshape(SKV, HG * DH).astype(jnp.bfloat16)
                for h in range(HG):
                    qh = q16[:, h * DH:(h + 1) * DH]
                    kh = kb[:, h * DH:(h + 1) * DH]
                    sc = lax.dot_general(
                        qh, kh, (((1,), (1,)), ((), ())),
                        preferred_element_type=jnp.float32)
                    sc = sc * 0.125 + bias
                    m = jnp.max(sc, axis=-1, keepdims=True)
                    w = jnp.exp(sc - m)
                    w = w / jnp.sum(w, axis=-1, keepdims=True)
                    ch = lax.dot(w.astype(jnp.bfloat16),
                                 vb[:, h * DH:(h + 1) * DH],
                                 preferred_element_type=jnp.float32)
                    ctx_buf[:, h * DH:(h + 1) * DH] = ch.astype(jnp.bfloat16)
                acc = lax.dot(ctx_buf[:, :], wo_g,
                              preferred_element_type=jnp.float32)
                if s == 0:
                    out_ref[b, :, :] = acc
                else:
                    out_ref[b, :, :] = out_ref[b, :, :] + acc

            if s < N_DEV - 1:
                rdma_wq.wait()
                rdma_wo.wait()

    return pl.pallas_call(
        body,
        out_shape=jax.ShapeDtypeStruct((B_LOC, SQ, D_MODEL), jnp.float32),
        in_specs=[
            pl.BlockSpec(memory_space=pltpu.VMEM),
            pl.BlockSpec(memory_space=pltpu.VMEM),
            pl.BlockSpec(memory_space=_ANY),
            pl.BlockSpec(memory_space=_ANY),
            pl.BlockSpec(memory_space=pltpu.VMEM),
        ],
        out_specs=pl.BlockSpec(memory_space=pltpu.VMEM),
        scratch_shapes=[
            pltpu.VMEM((N_DEV, D_MODEL, D_HID), jnp.bfloat16),
            pltpu.VMEM((N_DEV, D_HID, D_MODEL), jnp.bfloat16),
            pltpu.VMEM((B_LOC, SKV, HG, DH), jnp.float32),
            pltpu.VMEM((B_LOC, SKV, HG, DH), jnp.float32),
            pltpu.VMEM((SQ, HG * DH), jnp.bfloat16),
            pltpu.SemaphoreType.DMA((N_DEV - 1,)),
            pltpu.SemaphoreType.DMA((N_DEV - 1,)),
            pltpu.SemaphoreType.DMA((N_DEV - 1,)),
            pltpu.SemaphoreType.DMA((N_DEV - 1,)),
            pltpu.SemaphoreType.DMA((2,)),
        ],
        compiler_params=_CP(collective_id=0),
    )(x, Wq, K_ext, V_ext, Wo)


# baseline (device time: 204905 ns/iter reference)
import jax
import jax.numpy as jnp
from jax import lax
from jax.experimental import pallas as pl
from jax.experimental.pallas import tpu as pltpu

N_DEV = 4
B_LOC = 2
SQ = 512
SKV = 512
HG = 8
DH = 64
D_MODEL = 768
D_HID = HG * DH

_ANY = pl.ANY
_CP = getattr(pltpu, "CompilerParams", None)
if _CP is None:
    _CP = pltpu.TPUCompilerParams


def kernel(x, Wq, K_ext, V_ext, Wo):
    def body(x_ref, wq_ref, k_ref, v_ref, wo_ref, out_ref,
             wq_all, wo_all, kbuf, vbuf, ctx_buf,
             wq_send, wq_recv, wo_send, wo_recv, kv_sem):
        me = lax.axis_index("i")
        right = lax.rem(me + 1, N_DEV)
        left = lax.rem(me + N_DEV - 1, N_DEV)

        barrier = pltpu.get_barrier_semaphore()
        pl.semaphore_signal(barrier, inc=1, device_id=(left,),
                            device_id_type=pl.DeviceIdType.MESH)
        pl.semaphore_signal(barrier, inc=1, device_id=(right,),
                            device_id_type=pl.DeviceIdType.MESH)
        pl.semaphore_wait(barrier, 2)

        wq_all[0, :, :] = wq_ref[:, :].astype(jnp.bfloat16)
        wo_all[0, :, :] = wo_ref[:, :].astype(jnp.bfloat16)

        x16 = x_ref[:, :, :].astype(jnp.bfloat16)

        row = lax.broadcasted_iota(jnp.int32, (SQ, SKV), 0)
        col = lax.broadcasted_iota(jnp.int32, (SQ, SKV), 1)
        bias = jnp.where((row // 64) % 4 == (col // 64) % 4, 0.0, -1e9)
        bias = bias.astype(jnp.float32)

        for s in range(N_DEV):
            if s < N_DEV - 1:
                rdma_wq = pltpu.make_async_remote_copy(
                    src_ref=wq_all.at[s], dst_ref=wq_all.at[s + 1],
                    send_sem=wq_send.at[s], recv_sem=wq_recv.at[s],
                    device_id=(right,), device_id_type=pl.DeviceIdType.MESH)
                rdma_wo = pltpu.make_async_remote_copy(
                    src_ref=wo_all.at[s], dst_ref=wo_all.at[s + 1],
                    send_sem=wo_send.at[s], recv_sem=wo_recv.at[s],
                    device_id=(right,), device_id_type=pl.DeviceIdType.MESH)
                rdma_wq.start()
                rdma_wo.start()

            o = lax.rem(me - s + N_DEV, N_DEV)
            cp_k = pltpu.make_async_copy(
                k_ref.at[pl.ds(me * B_LOC, B_LOC), :, pl.ds(o * HG, HG), :],
                kbuf, kv_sem.at[0])
            cp_v = pltpu.make_async_copy(
                v_ref.at[pl.ds(me * B_LOC, B_LOC), :, pl.ds(o * HG, HG), :],
                vbuf, kv_sem.at[1])
            cp_k.start()
            cp_v.start()
            cp_k.wait()
            cp_v.wait()

            wq_g = wq_all[s]
            wo_g = wo_all[s]
            for b in range(B_LOC):
                q16 = lax.dot(x16[b], wq_g,
                              preferred_element_type=jnp.float32
                              ).astype(jnp.bfloat16)
                kb = kbuf[b].reshape(SKV, HG * DH).astype(jnp.bfloat16)
                vb = vbuf[b].reshape(SKV, HG * DH).astype(jnp.bfloat16)
                for h in range(HG):
                    qh = q16[:, h * DH:(h + 1) * DH]
                    kh = kb[:, h * DH:(h + 1) * DH]
                    sc = lax.dot_general(
                        qh, kh, (((1,), (1,)), ((), ())),
                        preferred_element_type=jnp.float32)
                    sc = sc * 0.125 + bias
                    m = jnp.max(sc, axis=-1, keepdims=True)
                    w = jnp.exp(sc - m)
                    w = w / jnp.sum(w, axis=-1, keepdims=True)
                    ch = lax.dot(w.astype(jnp.bfloat16),
                                 vb[:, h * DH:(h + 1) * DH],
                                 preferred_element_type=jnp.float32)
                    ctx_buf[:, h * DH:(h + 1) * DH] = ch.astype(jnp.bfloat16)
                acc = lax.dot(ctx_buf[:, :], wo_g,
                              preferred_element_type=jnp.float32)
                if s == 0:
                    out_ref[b, :, :] = acc
                else:
                    out_ref[b, :, :] = out_ref[b, :, :] + acc

            if s < N_DEV - 1:
                rdma_wq.wait()
                rdma_wo.wait()

    return pl.pallas_call(
        body,
        out_shape=jax.ShapeDtypeStruct((B_LOC, SQ, D_MODEL), jnp.float32),
        in_specs=[
            pl.BlockSpec(memory_space=pltpu.VMEM),
            pl.BlockSpec(memory_space=pltpu.VMEM),
            pl.BlockSpec(memory_space=_ANY),
            pl.BlockSpec(memory_space=_ANY),
            pl.BlockSpec(memory_space=pltpu.VMEM),
        ],
        out_specs=pl.BlockSpec(memory_space=pltpu.VMEM),
        scratch_shapes=[
            pltpu.VMEM((N_DEV, D_MODEL, D_HID), jnp.bfloat16),
            pltpu.VMEM((N_DEV, D_HID, D_MODEL), jnp.bfloat16),
            pltpu.VMEM((B_LOC, SKV, HG, DH), jnp.float32),
            pltpu.VMEM((B_LOC, SKV, HG, DH), jnp.float32),
            pltpu.VMEM((SQ, HG * DH), jnp.bfloat16),
            pltpu.SemaphoreType.DMA((N_DEV - 1,)),
            pltpu.SemaphoreType.DMA((N_DEV - 1,)),
            pltpu.SemaphoreType.DMA((N_DEV - 1,)),
            pltpu.SemaphoreType.DMA((N_DEV - 1,)),
            pltpu.SemaphoreType.DMA((2,)),
        ],
        compiler_params=_CP(collective_id=0),
    )(x, Wq, K_ext, V_ext, Wo)


# device time: 195467 ns/iter; 1.0483x vs baseline; 1.0483x over previous
import jax
import jax.numpy as jnp
from jax import lax
from jax.experimental import pallas as pl
from jax.experimental.pallas import tpu as pltpu

N_DEV = 4
B_LOC = 2
SQ = 512
SKV = 512
HG = 8
DH = 64
D_MODEL = 768
D_HID = HG * DH

_ANY = pl.ANY
_CP = getattr(pltpu, "CompilerParams", None)
if _CP is None:
    _CP = pltpu.TPUCompilerParams


def kernel(x, Wq, K_ext, V_ext, Wo):
    def body(x_ref, wq_ref, k_ref, v_ref, wo_ref, out_ref,
             wq_all, wo_all, kbuf, vbuf, ctx_buf, accp,
             wq_send, wq_recv, wo_send, wo_recv, kv_sem):
        me = lax.axis_index("i")
        right = lax.rem(me + 1, N_DEV)
        left = lax.rem(me + N_DEV - 1, N_DEV)

        barrier = pltpu.get_barrier_semaphore()
        pl.semaphore_signal(barrier, inc=1, device_id=(left,),
                            device_id_type=pl.DeviceIdType.MESH)
        pl.semaphore_signal(barrier, inc=1, device_id=(right,),
                            device_id_type=pl.DeviceIdType.MESH)
        pl.semaphore_wait(barrier, 2)

        wq_all[0, :, :] = wq_ref[:, :].astype(jnp.bfloat16)
        wo_all[0, :, :] = wo_ref[:, :].astype(jnp.bfloat16)

        x16 = x_ref[:, :, :].astype(jnp.bfloat16)

        def pack_rows(a):
            return jnp.concatenate(
                [a[64 * r + off:64 * r + off + 64]
                 for r in range(4) for off in (0, 256)], axis=0)

        xp = [pack_rows(x16[b]) for b in range(B_LOC)]

        for s in range(N_DEV):
            if s < N_DEV - 1:
                rdma_wq = pltpu.make_async_remote_copy(
                    src_ref=wq_all.at[s], dst_ref=wq_all.at[s + 1],
                    send_sem=wq_send.at[s], recv_sem=wq_recv.at[s],
                    device_id=(right,), device_id_type=pl.DeviceIdType.MESH)
                rdma_wo = pltpu.make_async_remote_copy(
                    src_ref=wo_all.at[s], dst_ref=wo_all.at[s + 1],
                    send_sem=wo_send.at[s], recv_sem=wo_recv.at[s],
                    device_id=(right,), device_id_type=pl.DeviceIdType.MESH)
                rdma_wq.start()
                rdma_wo.start()

            o = lax.rem(me - s + N_DEV, N_DEV)
            cp_k = pltpu.make_async_copy(
                k_ref.at[pl.ds(me * B_LOC, B_LOC), :, pl.ds(o * HG, HG), :],
                kbuf, kv_sem.at[0])
            cp_v = pltpu.make_async_copy(
                v_ref.at[pl.ds(me * B_LOC, B_LOC), :, pl.ds(o * HG, HG), :],
                vbuf, kv_sem.at[1])
            cp_k.start()
            cp_v.start()
            cp_k.wait()
            cp_v.wait()

            wq_g = wq_all[s]
            wo_g = wo_all[s]
            for b in range(B_LOC):
                qp = lax.dot(xp[b], wq_g,
                             preferred_element_type=jnp.float32
                             ).astype(jnp.bfloat16)
                kp = pack_rows(
                    kbuf[b].reshape(SKV, HG * DH).astype(jnp.bfloat16))
                vp = pack_rows(
                    vbuf[b].reshape(SKV, HG * DH).astype(jnp.bfloat16))
                for h in range(HG):
                    hc = slice(h * DH, (h + 1) * DH)
                    for r in range(4):
                        rows = slice(128 * r, 128 * r + 128)
                        sc = lax.dot_general(
                            qp[rows, hc], kp[rows, hc],
                            (((1,), (1,)), ((), ())),
                            preferred_element_type=jnp.float32)
                        w = jnp.exp(sc * 0.125)
                        sinv = 1.0 / jnp.sum(w, axis=-1, keepdims=True)
                        ch = lax.dot(w.astype(jnp.bfloat16), vp[rows, hc],
                                     preferred_element_type=jnp.float32)
                        ctx_buf[rows, hc] = (ch * sinv).astype(jnp.bfloat16)
                acc = lax.dot(ctx_buf[:, :], wo_g,
                              preferred_element_type=jnp.float32)
                if s == 0:
                    accp[b, :, :] = acc
                else:
                    accp[b, :, :] = accp[b, :, :] + acc

            if s < N_DEV - 1:
                rdma_wq.wait()
                rdma_wo.wait()

        for b in range(B_LOC):
            for r in range(4):
                out_ref[b, 64 * r:64 * r + 64, :] = \
                    accp[b, 128 * r:128 * r + 64, :]
                out_ref[b, 256 + 64 * r:256 + 64 * r + 64, :] = \
                    accp[b, 128 * r + 64:128 * r + 128, :]

    return pl.pallas_call(
        body,
        out_shape=jax.ShapeDtypeStruct((B_LOC, SQ, D_MODEL), jnp.float32),
        in_specs=[
            pl.BlockSpec(memory_space=pltpu.VMEM),
            pl.BlockSpec(memory_space=pltpu.VMEM),
            pl.BlockSpec(memory_space=_ANY),
            pl.BlockSpec(memory_space=_ANY),
            pl.BlockSpec(memory_space=pltpu.VMEM),
        ],
        out_specs=pl.BlockSpec(memory_space=pltpu.VMEM),
        scratch_shapes=[
            pltpu.VMEM((N_DEV, D_MODEL, D_HID), jnp.bfloat16),
            pltpu.VMEM((N_DEV, D_HID, D_MODEL), jnp.bfloat16),
            pltpu.VMEM((B_LOC, SKV, HG, DH), jnp.float32),
            pltpu.VMEM((B_LOC, SKV, HG, DH), jnp.float32),
            pltpu.VMEM((SQ, HG * DH), jnp.bfloat16),
            pltpu.VMEM((B_LOC, SQ, D_MODEL), jnp.float32),
            pltpu.SemaphoreType.DMA((N_DEV - 1,)),
            pltpu.SemaphoreType.DMA((N_DEV - 1,)),
            pltpu.SemaphoreType.DMA((N_DEV - 1,)),
            pltpu.SemaphoreType.DMA((N_DEV - 1,)),
            pltpu.SemaphoreType.DMA((2,)),
        ],
        compiler_params=_CP(collective_id=0),
    )(x, Wq, K_ext, V_ext, Wo)


# device time: 129171 ns/iter; 1.5863x vs baseline; 1.5132x over previous
import numpy as np
import jax
import jax.numpy as jnp
from jax import lax
from jax.experimental import pallas as pl
from jax.experimental.pallas import tpu as pltpu

N_DEV = 4
B_LOC = 2
SQ = 512
SKV = 512
HG = 8
DH = 64
D_MODEL = 768
D_HID = HG * DH

PERM = np.concatenate(
    [np.arange(64 * r + off, 64 * r + off + 64)
     for r in range(4) for off in (0, 256)])

_CP = getattr(pltpu, "CompilerParams", None)
if _CP is None:
    _CP = pltpu.TPUCompilerParams


def kernel(x, Wq, K_ext, V_ext, Wo):
    me = lax.axis_index("i")
    idx = jnp.mod(me - jnp.arange(N_DEV), N_DEV)

    xp = x[:, PERM, :].astype(jnp.bfloat16)

    def prep(t):
        t = lax.dynamic_slice_in_dim(t, me * B_LOC, B_LOC, 0)
        t = t.reshape(B_LOC, SKV, N_DEV, D_HID)[:, PERM]
        t = jnp.moveaxis(t, 2, 0).astype(jnp.bfloat16)
        return jnp.take(t, idx, axis=0)

    kp = prep(K_ext)
    vp = prep(V_ext)

    def body(xp_ref, wq_ref, kp_ref, vp_ref, wo_ref, out_ref,
             wq_all, wo_all, ctx_buf, accp,
             wq_send, wq_recv, wo_send, wo_recv):
        my = lax.axis_index("i")
        right = lax.rem(my + 1, N_DEV)
        left = lax.rem(my + N_DEV - 1, N_DEV)

        barrier = pltpu.get_barrier_semaphore()
        pl.semaphore_signal(barrier, inc=1, device_id=(left,),
                            device_id_type=pl.DeviceIdType.MESH)
        pl.semaphore_signal(barrier, inc=1, device_id=(right,),
                            device_id_type=pl.DeviceIdType.MESH)
        pl.semaphore_wait(barrier, 2)

        wq_all[0, :, :] = wq_ref[:, :].astype(jnp.bfloat16)
        wo_all[0, :, :] = wo_ref[:, :].astype(jnp.bfloat16)

        for s in range(N_DEV):
            if s < N_DEV - 1:
                rdma_wq = pltpu.make_async_remote_copy(
                    src_ref=wq_all.at[s], dst_ref=wq_all.at[s + 1],
                    send_sem=wq_send.at[s], recv_sem=wq_recv.at[s],
                    device_id=(right,), device_id_type=pl.DeviceIdType.MESH)
                rdma_wo = pltpu.make_async_remote_copy(
                    src_ref=wo_all.at[s], dst_ref=wo_all.at[s + 1],
                    send_sem=wo_send.at[s], recv_sem=wo_recv.at[s],
                    device_id=(right,), device_id_type=pl.DeviceIdType.MESH)
                rdma_wq.start()
                rdma_wo.start()

            wq_g = wq_all[s]
            wo_g = wo_all[s]
            for b in range(B_LOC):
                qp = lax.dot(xp_ref[b], wq_g,
                             preferred_element_type=jnp.float32
                             ).astype(jnp.bfloat16)
                kb = kp_ref[s, b]
                vb = vp_ref[s, b]
                for h in range(HG):
                    hc = slice(h * DH, (h + 1) * DH)
                    for r in range(4):
                        rows = slice(128 * r, 128 * r + 128)
                        sc = lax.dot_general(
                            qp[rows, hc], kb[rows, hc],
                            (((1,), (1,)), ((), ())),
                            preferred_element_type=jnp.float32)
                        w = jnp.exp(sc * 0.125)
                        sinv = 1.0 / jnp.sum(w, axis=-1, keepdims=True)
                        ch = lax.dot(w.astype(jnp.bfloat16), vb[rows, hc],
                                     preferred_element_type=jnp.float32)
                        ctx_buf[rows, hc] = (ch * sinv).astype(jnp.bfloat16)
                acc = lax.dot(ctx_buf[:, :], wo_g,
                              preferred_element_type=jnp.float32)
                if s == 0:
                    accp[b, :, :] = acc
                else:
                    accp[b, :, :] = accp[b, :, :] + acc

            if s < N_DEV - 1:
                rdma_wq.wait()
                rdma_wo.wait()

        for b in range(B_LOC):
            for r in range(4):
                out_ref[b, 64 * r:64 * r + 64, :] = \
                    accp[b, 128 * r:128 * r + 64, :]
                out_ref[b, 256 + 64 * r:256 + 64 * r + 64, :] = \
                    accp[b, 128 * r + 64:128 * r + 128, :]

    return pl.pallas_call(
        body,
        out_shape=jax.ShapeDtypeStruct((B_LOC, SQ, D_MODEL), jnp.float32),
        in_specs=[
            pl.BlockSpec(memory_space=pltpu.VMEM),
            pl.BlockSpec(memory_space=pltpu.VMEM),
            pl.BlockSpec(memory_space=pltpu.VMEM),
            pl.BlockSpec(memory_space=pltpu.VMEM),
            pl.BlockSpec(memory_space=pltpu.VMEM),
        ],
        out_specs=pl.BlockSpec(memory_space=pltpu.VMEM),
        scratch_shapes=[
            pltpu.VMEM((N_DEV, D_MODEL, D_HID), jnp.bfloat16),
            pltpu.VMEM((N_DEV, D_HID, D_MODEL), jnp.bfloat16),
            pltpu.VMEM((SQ, HG * DH), jnp.bfloat16),
            pltpu.VMEM((B_LOC, SQ, D_MODEL), jnp.float32),
            pltpu.SemaphoreType.DMA((N_DEV - 1,)),
            pltpu.SemaphoreType.DMA((N_DEV - 1,)),
            pltpu.SemaphoreType.DMA((N_DEV - 1,)),
            pltpu.SemaphoreType.DMA((N_DEV - 1,)),
        ],
        compiler_params=_CP(collective_id=0),
    )(xp, Wq, kp, vp, Wo)


# device time: 57458 ns/iter; 3.5662x vs baseline; 2.2481x over previous
import jax
import jax.numpy as jnp
from jax import lax
from jax.experimental import pallas as pl
from jax.experimental.pallas import tpu as pltpu

N_DEV = 4
B_LOC = 2
SQ = 512
SKV = 512
HG = 8
DH = 64
D_MODEL = 768
D_HID = HG * DH

_CP = getattr(pltpu, "CompilerParams", None)
if _CP is None:
    _CP = pltpu.TPUCompilerParams

_PACK_SLICES = [(64 * r + off, 64 * r + off + 64)
                for r in range(4) for off in (0, 256)]


def _pack_rows(a):
    return jnp.concatenate([a[lo:hi] for lo, hi in _PACK_SLICES], axis=0)


def _pack_cols(a):
    return jnp.concatenate([a[:, lo:hi] for lo, hi in _PACK_SLICES], axis=1)


def kernel(x, Wq, K_ext, V_ext, Wo):
    kt = jnp.transpose(K_ext, (0, 2, 3, 1))
    vt = jnp.transpose(V_ext, (0, 2, 3, 1))

    def body(x_ref, wq_ref, kt_ref, vt_ref, wo_ref, out_ref,
             wq_all, wo_all, ktbuf, vtbuf, ctx_buf, accp,
             send_s, recv_s, kv_sem):
        me = lax.axis_index("i")
        right = lax.rem(me + 1, N_DEV)
        left = lax.rem(me + N_DEV - 1, N_DEV)

        barrier = pltpu.get_barrier_semaphore()
        pl.semaphore_signal(barrier, inc=1, device_id=(left,),
                            device_id_type=pl.DeviceIdType.MESH)
        pl.semaphore_signal(barrier, inc=1, device_id=(right,),
                            device_id_type=pl.DeviceIdType.MESH)
        pl.semaphore_wait(barrier, 2)

        wq_all[0, :, :] = wq_ref[:, :].astype(jnp.bfloat16)
        wo_all[0, :, :] = wo_ref[:, :].astype(jnp.bfloat16)

        xp = [_pack_rows(x_ref[b].astype(jnp.bfloat16))
              for b in range(B_LOC)]

        def mk(buf, src_ix, dst_ix, sem_ix, dev):
            return pltpu.make_async_remote_copy(
                src_ref=buf.at[src_ix] if not isinstance(src_ix, tuple)
                else buf.at[src_ix[0], src_ix[1]],
                dst_ref=buf.at[dst_ix] if not isinstance(dst_ix, tuple)
                else buf.at[dst_ix[0], dst_ix[1]],
                send_sem=send_s.at[sem_ix], recv_sem=recv_s.at[sem_ix],
                device_id=(dev,), device_id_type=pl.DeviceIdType.MESH)

        def compute_step(s):
            off = (0, N_DEV - 1, 1, 2)[s]
            o = lax.rem(me + off, N_DEV)
            cp_k = pltpu.make_async_copy(
                kt_ref.at[pl.ds(me * B_LOC, B_LOC), pl.ds(o * HG, HG), :, :],
                ktbuf, kv_sem.at[0])
            cp_v = pltpu.make_async_copy(
                vt_ref.at[pl.ds(me * B_LOC, B_LOC), pl.ds(o * HG, HG), :, :],
                vtbuf, kv_sem.at[1])
            cp_k.start()
            cp_v.start()
            cp_k.wait()
            cp_v.wait()

            wq_g = wq_all[s]
            wo_g = wo_all[s]
            for b in range(B_LOC):
                qp = (lax.dot(xp[b], wq_g,
                              preferred_element_type=jnp.float32)
                      * 0.125).astype(jnp.bfloat16)
                ktp = _pack_cols(
                    ktbuf[b].reshape(HG * DH, SKV).astype(jnp.bfloat16))
                vtp = _pack_cols(
                    vtbuf[b].reshape(HG * DH, SKV).astype(jnp.bfloat16))
                for h in range(HG):
                    hr = slice(h * DH, (h + 1) * DH)
                    for r in range(4):
                        rows = slice(128 * r, 128 * r + 128)
                        sc = lax.dot(qp[rows, hr], ktp[hr, rows],
                                     preferred_element_type=jnp.float32)
                        w = jnp.exp(sc)
                        sinv = 1.0 / jnp.sum(w, axis=-1, keepdims=True)
                        ch = lax.dot_general(
                            w.astype(jnp.bfloat16), vtp[hr, rows],
                            (((1,), (1,)), ((), ())),
                            preferred_element_type=jnp.float32)
                        ctx_buf[rows, hr] = (ch * sinv).astype(jnp.bfloat16)
                acc = lax.dot(ctx_buf[:, :], wo_g,
                              preferred_element_type=jnp.float32)
                if s == 0:
                    accp[b, :, :] = acc
                else:
                    accp[b, :, :] = accp[b, :, :] + acc

        ar_wq = mk(wq_all, 0, 1, 0, right)
        ar_wo = mk(wo_all, 0, 1, 1, right)
        al_wq = mk(wq_all, 0, 2, 2, left)
        al_wo = mk(wo_all, 0, 2, 3, left)
        ar_wq.start()
        ar_wo.start()
        al_wq.start()
        al_wo.start()

        compute_step(0)

        ar_wq.wait()
        ar_wo.wait()
        br_wq = mk(wq_all, (1, pl.ds(0, 384)), (3, pl.ds(0, 384)), 4, right)
        br_wo = mk(wo_all, (1, pl.ds(0, 256)), (3, pl.ds(0, 256)), 5, right)
        br_wq.start()
        br_wo.start()

        compute_step(1)

        al_wq.wait()
        al_wo.wait()
        bl_wq = mk(wq_all, (2, pl.ds(384, 384)), (3, pl.ds(384, 384)), 6, left)
        bl_wo = mk(wo_all, (2, pl.ds(256, 256)), (3, pl.ds(256, 256)), 7, left)
        bl_wq.start()
        bl_wo.start()

        compute_step(2)

        br_wq.wait()
        br_wo.wait()
        bl_wq.wait()
        bl_wo.wait()

        compute_step(3)

        for b in range(B_LOC):
            for r in range(4):
                out_ref[b, 64 * r:64 * r + 64, :] = \
                    accp[b, 128 * r:128 * r + 64, :]
                out_ref[b, 256 + 64 * r:256 + 64 * r + 64, :] = \
                    accp[b, 128 * r + 64:128 * r + 128, :]

    return pl.pallas_call(
        body,
        out_shape=jax.ShapeDtypeStruct((B_LOC, SQ, D_MODEL), jnp.float32),
        in_specs=[
            pl.BlockSpec(memory_space=pltpu.VMEM),
            pl.BlockSpec(memory_space=pltpu.VMEM),
            pl.BlockSpec(memory_space=pl.ANY),
            pl.BlockSpec(memory_space=pl.ANY),
            pl.BlockSpec(memory_space=pltpu.VMEM),
        ],
        out_specs=pl.BlockSpec(memory_space=pltpu.VMEM),
        scratch_shapes=[
            pltpu.VMEM((N_DEV, D_MODEL, D_HID), jnp.bfloat16),
            pltpu.VMEM((N_DEV, D_HID, D_MODEL), jnp.bfloat16),
            pltpu.VMEM((B_LOC, HG, DH, SKV), jnp.float32),
            pltpu.VMEM((B_LOC, HG, DH, SKV), jnp.float32),
            pltpu.VMEM((SQ, HG * DH), jnp.bfloat16),
            pltpu.VMEM((B_LOC, SQ, D_MODEL), jnp.float32),
            pltpu.SemaphoreType.DMA((8,)),
            pltpu.SemaphoreType.DMA((8,)),
            pltpu.SemaphoreType.DMA((2,)),
        ],
        compiler_params=_CP(collective_id=0),
    )(x, Wq, kt, vt, Wo)


# device time: 44213 ns/iter; 4.6345x vs baseline; 1.2996x over previous
import jax
import jax.numpy as jnp
from jax import lax
from jax.experimental import pallas as pl
from jax.experimental.pallas import tpu as pltpu

N_DEV = 4
B_LOC = 2
SQ = 512
SKV = 512
HG = 8
DH = 64
D_MODEL = 768
D_HID = HG * DH

_CP = getattr(pltpu, "CompilerParams", None)
if _CP is None:
    _CP = pltpu.TPUCompilerParams

_PACK_SLICES = [(64 * r + off, 64 * r + off + 64)
                for r in range(4) for off in (0, 256)]


def _pack_rows(a):
    return jnp.concatenate([a[lo:hi] for lo, hi in _PACK_SLICES], axis=0)


def _pack_cols(a):
    return jnp.concatenate([a[:, lo:hi] for lo, hi in _PACK_SLICES], axis=1)


def kernel(x, Wq, K_ext, V_ext, Wo):
    kt = jnp.transpose(K_ext, (0, 2, 3, 1))
    vt = jnp.transpose(V_ext, (0, 2, 3, 1))

    def body(x_ref, wq_ref, kt_ref, vt_ref, wo_ref, out_ref,
             wq_all, wo_all, ktbuf, vtbuf, ctx_buf, accp,
             send_s, recv_s, kv_sem):
        me = lax.axis_index("i")
        right = lax.rem(me + 1, N_DEV)
        left = lax.rem(me + N_DEV - 1, N_DEV)

        barrier = pltpu.get_barrier_semaphore()
        pl.semaphore_signal(barrier, inc=1, device_id=(left,),
                            device_id_type=pl.DeviceIdType.MESH)
        pl.semaphore_signal(barrier, inc=1, device_id=(right,),
                            device_id_type=pl.DeviceIdType.MESH)
        pl.semaphore_wait(barrier, 2)

        wq_all[0, :, :] = wq_ref[:, :].astype(jnp.bfloat16)
        wo_all[0, :, :] = wo_ref[:, :].astype(jnp.bfloat16)

        xp_cat = jnp.concatenate(
            [_pack_rows(x_ref[b].astype(jnp.bfloat16))
             for b in range(B_LOC)], axis=0)

        def mk(buf, src_ix, dst_ix, sem_ix, dev):
            return pltpu.make_async_remote_copy(
                src_ref=buf.at[src_ix] if not isinstance(src_ix, tuple)
                else buf.at[src_ix[0], src_ix[1]],
                dst_ref=buf.at[dst_ix] if not isinstance(dst_ix, tuple)
                else buf.at[dst_ix[0], dst_ix[1]],
                send_sem=send_s.at[sem_ix], recv_sem=recv_s.at[sem_ix],
                device_id=(dev,), device_id_type=pl.DeviceIdType.MESH)

        def kv_copies(s, slot):
            off = (0, N_DEV - 1, 1, 2)[s]
            o = lax.rem(me + off, N_DEV)
            cp_k = pltpu.make_async_copy(
                kt_ref.at[pl.ds(me * B_LOC, B_LOC), pl.ds(o * HG, HG), :, :],
                ktbuf.at[slot], kv_sem.at[slot, 0])
            cp_v = pltpu.make_async_copy(
                vt_ref.at[pl.ds(me * B_LOC, B_LOC), pl.ds(o * HG, HG), :, :],
                vtbuf.at[slot], kv_sem.at[slot, 1])
            return cp_k, cp_v

        def attn_phase(s):
            slot = s % 2
            cp_k, cp_v = kv_copies(s, slot)
            cp_k.wait()
            cp_v.wait()
            if s < N_DEV - 1:
                nx_k, nx_v = kv_copies(s + 1, (s + 1) % 2)
                nx_k.start()
                nx_v.start()
            wq_g = wq_all[s]
            qp = (lax.dot(xp_cat, wq_g,
                          preferred_element_type=jnp.float32)
                  * 0.125).astype(jnp.bfloat16)
            for b in range(B_LOC):
                ktp = _pack_cols(
                    ktbuf[slot, b].reshape(HG * DH, SKV).astype(jnp.bfloat16))
                vtp = _pack_cols(
                    vtbuf[slot, b].reshape(HG * DH, SKV).astype(jnp.bfloat16))
                for h in range(HG):
                    hr = slice(h * DH, (h + 1) * DH)
                    for r in range(4):
                        rows = slice(128 * r, 128 * r + 128)
                        qrows = slice(512 * b + 128 * r, 512 * b + 128 * r + 128)
                        sc = lax.dot(qp[qrows, hr], ktp[hr, rows],
                                     preferred_element_type=jnp.float32)
                        w = jnp.exp(sc)
                        sinv = 1.0 / jnp.sum(w, axis=-1, keepdims=True)
                        ch = lax.dot_general(
                            w.astype(jnp.bfloat16), vtp[hr, rows],
                            (((1,), (1,)), ((), ())),
                            preferred_element_type=jnp.float32)
                        ctx_buf[qrows, hr] = (ch * sinv).astype(jnp.bfloat16)

        def out_phase(s):
            acc = lax.dot(ctx_buf[:, :], wo_all[s],
                          preferred_element_type=jnp.float32)
            for b in range(B_LOC):
                a = acc[512 * b:512 * (b + 1)]
                if s == 0:
                    accp[b, :, :] = a
                else:
                    accp[b, :, :] = accp[b, :, :] + a

        cp_k0, cp_v0 = kv_copies(0, 0)
        cp_k0.start()
        cp_v0.start()

        ar_wq = mk(wq_all, 0, 1, 0, right)
        ar_wo = mk(wo_all, 0, 1, 1, right)
        al_wq = mk(wq_all, 0, 2, 2, left)
        al_wo = mk(wo_all, 0, 2, 3, left)
        ar_wq.start()
        ar_wo.start()
        al_wq.start()
        al_wo.start()

        attn_phase(0)
        out_phase(0)

        ar_wq.wait()
        br_wq = mk(wq_all, (1, pl.ds(0, 384)), (3, pl.ds(0, 384)), 4, right)
        br_wq.start()
        attn_phase(1)
        ar_wo.wait()
        br_wo = mk(wo_all, (1, pl.ds(0, 256)), (3, pl.ds(0, 256)), 5, right)
        br_wo.start()
        out_phase(1)

        al_wq.wait()
        bl_wq = mk(wq_all, (2, pl.ds(384, 384)), (3, pl.ds(384, 384)), 6, left)
        bl_wq.start()
        attn_phase(2)
        al_wo.wait()
        bl_wo = mk(wo_all, (2, pl.ds(256, 256)), (3, pl.ds(256, 256)), 7, left)
        bl_wo.start()
        out_phase(2)

        br_wq.wait()
        bl_wq.wait()
        attn_phase(3)
        br_wo.wait()
        bl_wo.wait()
        out_phase(3)

        for b in range(B_LOC):
            for r in range(4):
                out_ref[b, 64 * r:64 * r + 64, :] = \
                    accp[b, 128 * r:128 * r + 64, :]
                out_ref[b, 256 + 64 * r:256 + 64 * r + 64, :] = \
                    accp[b, 128 * r + 64:128 * r + 128, :]

    return pl.pallas_call(
        body,
        out_shape=jax.ShapeDtypeStruct((B_LOC, SQ, D_MODEL), jnp.float32),
        in_specs=[
            pl.BlockSpec(memory_space=pltpu.VMEM),
            pl.BlockSpec(memory_space=pltpu.VMEM),
            pl.BlockSpec(memory_space=pl.ANY),
            pl.BlockSpec(memory_space=pl.ANY),
            pl.BlockSpec(memory_space=pltpu.VMEM),
        ],
        out_specs=pl.BlockSpec(memory_space=pltpu.VMEM),
        scratch_shapes=[
            pltpu.VMEM((N_DEV, D_MODEL, D_HID), jnp.bfloat16),
            pltpu.VMEM((N_DEV, D_HID, D_MODEL), jnp.bfloat16),
            pltpu.VMEM((2, B_LOC, HG, DH, SKV), jnp.float32),
            pltpu.VMEM((2, B_LOC, HG, DH, SKV), jnp.float32),
            pltpu.VMEM((B_LOC * SQ, HG * DH), jnp.bfloat16),
            pltpu.VMEM((B_LOC, SQ, D_MODEL), jnp.float32),
            pltpu.SemaphoreType.DMA((8,)),
            pltpu.SemaphoreType.DMA((8,)),
            pltpu.SemaphoreType.DMA((2, 2)),
        ],
        compiler_params=_CP(collective_id=0),
    )(x, Wq, kt, vt, Wo)
